# baseline (speedup 1.0000x reference)
import functools

import jax
import jax.numpy as jnp
from jax import lax
from jax.experimental import pallas as pl
from jax.experimental.pallas import tpu as pltpu
from jax.experimental.pallas import tpu_sc as plsc

DIM = 64
NC = 2
NS = 16
NW = NC * NS

LANES = 16


def _transpose_tile(win, tout):
    lane = lax.iota(jnp.int32, LANES)
    for q in range(64):
        for v in range(8):
            half = v // 4
            zvec = (16 * v) % 64 + lane
            cvec = jnp.full((LANES,), 2 * q + half, jnp.int32)
            tout[q, pl.ds(16 * v, LANES)] = plsc.load_gather(
                win, [zvec, cvec])


def _make_transpose_kernel(n_tok, n_tiles):
    mesh = plsc.VectorSubcoreMesh(core_axis_name="c", subcore_axis_name="s")
    base_tiles = n_tiles // NW
    extra_w = n_tiles - base_tiles * NW

    @functools.partial(
        pl.kernel,
        out_type=jax.ShapeDtypeStruct((n_tok // 2, 128), jnp.float32),
        mesh=mesh,
        scratch_types=[
            pltpu.VMEM((2, DIM, 128), jnp.float32),
            pltpu.VMEM((2, 64, 128), jnp.float32),
            pltpu.SemaphoreType.DMA((2,)),
            pltpu.SemaphoreType.DMA((2,)),
        ],
        compiler_params=pltpu.CompilerParams(needs_layout_passes=False),
    )
    def transpose_kernel(tableT, tail, scratch, win, tout, rsem, wsem):
        wid = lax.axis_index("s") * NC + lax.axis_index("c")

        def fire_read(k, b):
            pltpu.async_copy(
                tableT.at[:, pl.ds((k * NW + wid) * 128, 128)], win.at[b],
                rsem.at[b])

        def wait_read(k, b):
            pltpu.make_async_copy(
                tableT.at[:, pl.ds((k * NW + wid) * 128, 128)], win.at[b],
                rsem.at[b]).wait()

        def fire_write(k, b):
            pltpu.async_copy(
                tout.at[b], scratch.at[pl.ds((k * NW + wid) * 64, 64)],
                wsem.at[b])

        def wait_write(k, b):
            pltpu.make_async_copy(
                tout.at[b], scratch.at[pl.ds((k * NW + wid) * 64, 64)],
                wsem.at[b]).wait()

        fire_read(0, 0)
        n_pair = base_tiles // 2

        def body(s, _):
            k = s * 2
            fire_read(k + 1, 1)
            wait_read(k, 0)

            @pl.when(s >= 1)
            def _():
                wait_write(k - 2, 0)

            _transpose_tile(win.at[0], tout.at[0])
            fire_write(k, 0)

            @pl.when(s + 1 < n_pair)
            def _():
                fire_read(k + 2, 0)

            wait_read(k + 1, 1)

            @pl.when(s >= 1)
            def _():
                wait_write(k - 1, 1)

            _transpose_tile(win.at[1], tout.at[1])
            fire_write(k + 1, 1)
            return ()

        lax.fori_loop(0, n_pair, body, (), unroll=False)
        wait_write(base_tiles - 2, 0)
        wait_write(base_tiles - 1, 1)

        @pl.when(wid < extra_w)
        def _():
            fire_read(base_tiles, 0)
            wait_read(base_tiles, 0)
            _transpose_tile(win.at[0], tout.at[0])
            fire_write(base_tiles, 0)
            wait_write(base_tiles, 0)

        @pl.when(wid == 0)
        def _():
            pltpu.sync_copy(tail, tout.at[0].at[pl.ds(0, 32)])
            pltpu.sync_copy(
                tout.at[0].at[pl.ds(0, 32)],
                scratch.at[pl.ds(n_tok // 2 - 32, 32)])

    return transpose_kernel


def _make_gather_kernel(batch, fields):
    n_flat = batch * fields
    tok_pw = n_flat // NW
    i_pw = batch // NW
    G = 256
    n_h = i_pw // G
    n_g = fields * n_h

    mesh = plsc.VectorSubcoreMesh(core_axis_name="c", subcore_axis_name="s")

    @functools.partial(
        pl.kernel,
        out_type=jax.ShapeDtypeStruct((fields, DIM, batch), jnp.float32),
        mesh=mesh,
        scratch_types=[
            pltpu.VMEM((tok_pw,), jnp.int32),
            pltpu.VMEM((2, 2, 128), jnp.int32),
            pltpu.VMEM((2, G), jnp.int32),
            pltpu.VMEM((2, G, 128), jnp.float32),
            pltpu.VMEM((2, DIM, G), jnp.float32),
            pltpu.SemaphoreType.DMA((2, 2)),
            pltpu.SemaphoreType.DMA((2,)),
        ],
        compiler_params=pltpu.CompilerParams(needs_layout_passes=False),
    )
    def gather_kernel(idx_hbm, scratch_hbm, out_hbm, idx_v, rows_v, half_v,
                      grows, outw, gsem, osem):
        wid = lax.axis_index("s") * NC + lax.axis_index("c")
        pltpu.sync_copy(idx_hbm.at[pl.ds(wid * tok_pw, tok_pw)], idx_v)

        lane = lax.iota(jnp.int32, LANES)
        lanef = lane * fields

        def build_rows(g, b):
            j = g // n_h
            h = lax.rem(g, n_h)
            base = (h * G) * fields + j
            for s in range(2):
                def m_body(m, _):
                    pos = lanef + (base + (s * 8 + m) * LANES * fields)
                    gidx = plsc.load_gather(idx_v, [pos])
                    rows_v[b, s, pl.ds(m * LANES, LANES)] = (
                        lax.shift_right_logical(gidx, 1))
                    half_v[b, pl.ds((s * 8 + m) * LANES, LANES)] = (
                        lax.bitwise_and(gidx, 1))
                    return ()
                lax.fori_loop(0, 8, m_body, (), unroll=True)

        def fire_gathers(b):
            for s in range(2):
                pltpu.async_copy(
                    scratch_hbm.at[rows_v.at[b].at[s]],
                    grows.at[b].at[pl.ds(s * 128, 128)],
                    gsem.at[b, s])

        def wait_gathers(b):
            for s in range(2):
                pltpu.make_async_copy(
                    scratch_hbm.at[rows_v.at[b].at[s]],
                    grows.at[b].at[pl.ds(s * 128, 128)],
                    gsem.at[b, s]).wait()

        def transpose_sel(b):
            def m_body(m, _):
                kvec = m * LANES + lane
                hvec = half_v[b, pl.ds(m * LANES, LANES)] * DIM
                for d in range(DIM):
                    outw[b, d, pl.ds(m * LANES, LANES)] = plsc.load_gather(
                        grows.at[b], [kvec, hvec + d])
                return ()
            lax.fori_loop(0, G // LANES, m_body, (), unroll=False)

        def out_ref(g):
            j = g // n_h
            h = lax.rem(g, n_h)
            i0 = wid * i_pw + h * G
            return out_hbm.at[j].at[:, pl.ds(i0, G)]

        def fire_out(g, b):
            pltpu.async_copy(outw.at[b], out_ref(g), osem.at[b])

        def wait_out(g, b):
            pltpu.make_async_copy(outw.at[b], out_ref(g), osem.at[b]).wait()

        build_rows(0, 0)
        fire_gathers(0)
        n_pair = n_g // 2

        def body(p, _):
            g = p * 2
            build_rows(g + 1, 1)
            fire_gathers(1)
            wait_gathers(0)

            @pl.when(p >= 1)
            def _():
                wait_out(g - 2, 0)

            transpose_sel(0)
            fire_out(g, 0)

            @pl.when(p + 1 < n_pair)
            def _():
                build_rows(g + 2, 0)
                fire_gathers(0)

            wait_gathers(1)

            @pl.when(p >= 1)
            def _():
                wait_out(g - 1, 1)

            transpose_sel(1)
            fire_out(g + 1, 1)
            return ()

        lax.fori_loop(0, n_pair, body, (), unroll=False)
        wait_out(n_g - 2, 0)
        wait_out(n_g - 1, 1)

    return gather_kernel


@functools.partial(jax.jit, static_argnames=("batch", "fields", "n_tok"))
def _impl(actions, table, batch, fields, n_tok):
    n_tiles = (n_tok - DIM) // 128
    tail = table[n_tok - DIM:].reshape(32, 128)
    scratch = _make_transpose_kernel(n_tok, n_tiles)(table.T, tail)
    idx = actions.reshape(-1)
    out = _make_gather_kernel(batch, fields)(idx, scratch)
    return out.transpose(2, 0, 1)


def kernel(actions, embed_table):
    batch, fields = actions.shape
    n_tok = embed_table.shape[0]
    return _impl(actions, embed_table, batch, fields, n_tok)


# speedup vs baseline: 7.2077x; 7.2077x over previous
import functools

import jax
import jax.numpy as jnp
from jax import lax
from jax.experimental import pallas as pl
from jax.experimental.pallas import tpu as pltpu
from jax.experimental.pallas import tpu_sc as plsc

DIM = 64
NC = 2
NS = 16
NW = NC * NS

LANES = 16


def _transpose_tile(win, tout):
    lane = lax.iota(jnp.int32, LANES)
    zvecs = [(16 * v) % 64 + lane for v in range(8)]

    @functools.partial(plsc.parallel_loop, 0, 64, unroll=8)
    def _(q):
        for v in range(8):
            cvec = jnp.full((LANES,), 2 * q + v // 4, jnp.int32)
            tout[q, pl.ds(16 * v, LANES)] = plsc.load_gather(
                win, [zvecs[v], cvec])


def _make_transpose_kernel(n_tok, n_tiles):
    mesh = plsc.VectorSubcoreMesh(core_axis_name="c", subcore_axis_name="s")
    base_tiles = n_tiles // NW
    extra_w = n_tiles - base_tiles * NW

    @functools.partial(
        pl.kernel,
        out_type=jax.ShapeDtypeStruct((n_tok // 2, 128), jnp.float32),
        mesh=mesh,
        scratch_types=[
            pltpu.VMEM((2, DIM, 128), jnp.float32),
            pltpu.VMEM((2, 64, 128), jnp.float32),
            pltpu.SemaphoreType.DMA((2,)),
            pltpu.SemaphoreType.DMA((2,)),
        ],
        compiler_params=pltpu.CompilerParams(
            needs_layout_passes=False, disable_bounds_checks=True),
    )
    def transpose_kernel(tableT, tail, scratch, win, tout, rsem, wsem):
        wid = lax.axis_index("s") * NC + lax.axis_index("c")

        def fire_read(k, b):
            pltpu.async_copy(
                tableT.at[:, pl.ds((k * NW + wid) * 128, 128)], win.at[b],
                rsem.at[b])

        def wait_read(k, b):
            pltpu.make_async_copy(
                tableT.at[:, pl.ds((k * NW + wid) * 128, 128)], win.at[b],
                rsem.at[b]).wait()

        def fire_write(k, b):
            pltpu.async_copy(
                tout.at[b], scratch.at[pl.ds((k * NW + wid) * 64, 64)],
                wsem.at[b])

        def wait_write(k, b):
            pltpu.make_async_copy(
                tout.at[b], scratch.at[pl.ds((k * NW + wid) * 64, 64)],
                wsem.at[b]).wait()

        fire_read(0, 0)
        n_pair = base_tiles // 2

        def body(s, _):
            k = s * 2
            fire_read(k + 1, 1)
            wait_read(k, 0)

            @pl.when(s >= 1)
            def _():
                wait_write(k - 2, 0)

            _transpose_tile(win.at[0], tout.at[0])
            fire_write(k, 0)

            @pl.when(s + 1 < n_pair)
            def _():
                fire_read(k + 2, 0)

            wait_read(k + 1, 1)

            @pl.when(s >= 1)
            def _():
                wait_write(k - 1, 1)

            _transpose_tile(win.at[1], tout.at[1])
            fire_write(k + 1, 1)
            return ()

        lax.fori_loop(0, n_pair, body, (), unroll=False)
        wait_write(base_tiles - 2, 0)
        wait_write(base_tiles - 1, 1)

        @pl.when(wid < extra_w)
        def _():
            fire_read(base_tiles, 0)
            wait_read(base_tiles, 0)
            _transpose_tile(win.at[0], tout.at[0])
            fire_write(base_tiles, 0)
            wait_write(base_tiles, 0)

        @pl.when(wid == 0)
        def _():
            pltpu.sync_copy(tail, tout.at[0].at[pl.ds(0, 32)])
            pltpu.sync_copy(
                tout.at[0].at[pl.ds(0, 32)],
                scratch.at[pl.ds(n_tok // 2 - 32, 32)])

    return transpose_kernel


def _make_gather_kernel(batch, fields):
    n_flat = batch * fields
    tok_pw = n_flat // NW
    i_pw = batch // NW
    G = 256
    n_h = i_pw // G
    n_g = fields * n_h

    mesh = plsc.VectorSubcoreMesh(core_axis_name="c", subcore_axis_name="s")

    @functools.partial(
        pl.kernel,
        out_type=jax.ShapeDtypeStruct((fields, DIM, batch), jnp.float32),
        mesh=mesh,
        scratch_types=[
            pltpu.VMEM((tok_pw,), jnp.int32),
            pltpu.VMEM((2, 2, 128), jnp.int32),
            pltpu.VMEM((2, G), jnp.int32),
            pltpu.VMEM((2, G, 128), jnp.float32),
            pltpu.VMEM((2, DIM, G), jnp.float32),
            pltpu.SemaphoreType.DMA((2, 2)),
            pltpu.SemaphoreType.DMA((2,)),
        ],
        compiler_params=pltpu.CompilerParams(
            needs_layout_passes=False, disable_bounds_checks=True),
    )
    def gather_kernel(idx_hbm, scratch_hbm, out_hbm, idx_v, rows_v, half_v,
                      grows, outw, gsem, osem):
        wid = lax.axis_index("s") * NC + lax.axis_index("c")
        pltpu.sync_copy(idx_hbm.at[pl.ds(wid * tok_pw, tok_pw)], idx_v)

        lane = lax.iota(jnp.int32, LANES)
        lanef = lane * fields

        def build_rows(g, b):
            j = g // n_h
            h = lax.rem(g, n_h)
            base = (h * G) * fields + j
            for s in range(2):
                def m_body(m, _):
                    pos = lanef + (base + (s * 8 + m) * LANES * fields)
                    gidx = plsc.load_gather(idx_v, [pos])
                    rows_v[b, s, pl.ds(m * LANES, LANES)] = (
                        lax.shift_right_logical(gidx, 1))
                    half_v[b, pl.ds((s * 8 + m) * LANES, LANES)] = (
                        lax.bitwise_and(gidx, 1))
                    return ()
                lax.fori_loop(0, 8, m_body, (), unroll=True)

        def fire_gathers(b):
            for s in range(2):
                pltpu.async_copy(
                    scratch_hbm.at[rows_v.at[b].at[s]],
                    grows.at[b].at[pl.ds(s * 128, 128)],
                    gsem.at[b, s])

        def wait_gathers(b):
            for s in range(2):
                pltpu.make_async_copy(
                    scratch_hbm.at[rows_v.at[b].at[s]],
                    grows.at[b].at[pl.ds(s * 128, 128)],
                    gsem.at[b, s]).wait()

        def transpose_sel(b):
            @functools.partial(plsc.parallel_loop, 0, G // LANES, unroll=2)
            def _(m):
                kvec = m * LANES + lane
                hvec = half_v[b, pl.ds(m * LANES, LANES)] * DIM
                for d in range(DIM):
                    outw[b, d, pl.ds(m * LANES, LANES)] = plsc.load_gather(
                        grows.at[b], [kvec, hvec + d])

        def out_ref(g):
            j = g // n_h
            h = lax.rem(g, n_h)
            i0 = wid * i_pw + h * G
            return out_hbm.at[j].at[:, pl.ds(i0, G)]

        def fire_out(g, b):
            pltpu.async_copy(outw.at[b], out_ref(g), osem.at[b])

        def wait_out(g, b):
            pltpu.make_async_copy(outw.at[b], out_ref(g), osem.at[b]).wait()

        build_rows(0, 0)
        fire_gathers(0)
        n_pair = n_g // 2

        def body(p, _):
            g = p * 2
            build_rows(g + 1, 1)
            fire_gathers(1)
            wait_gathers(0)

            @pl.when(p >= 1)
            def _():
                wait_out(g - 2, 0)

            transpose_sel(0)
            fire_out(g, 0)

            @pl.when(p + 1 < n_pair)
            def _():
                build_rows(g + 2, 0)
                fire_gathers(0)

            wait_gathers(1)

            @pl.when(p >= 1)
            def _():
                wait_out(g - 1, 1)

            transpose_sel(1)
            fire_out(g + 1, 1)
            return ()

        lax.fori_loop(0, n_pair, body, (), unroll=False)
        wait_out(n_g - 2, 0)
        wait_out(n_g - 1, 1)

    return gather_kernel


@functools.partial(jax.jit, static_argnames=("batch", "fields", "n_tok"))
def _impl(actions, table, batch, fields, n_tok):
    n_tiles = (n_tok - DIM) // 128
    tail = table[n_tok - DIM:].reshape(32, 128)
    scratch = _make_transpose_kernel(n_tok, n_tiles)(table.T, tail)
    idx = actions.reshape(-1)
    out = _make_gather_kernel(batch, fields)(idx, scratch)
    return out.transpose(2, 0, 1)


def kernel(actions, embed_table):
    batch, fields = actions.shape
    n_tok = embed_table.shape[0]
    return _impl(actions, embed_table, batch, fields, n_tok)
